# baseline (speedup 1.0000x reference)
import functools

import jax
import jax.numpy as jnp
from jax import lax
from jax.experimental import pallas as pl
from jax.experimental.pallas import tpu as pltpu
from jax.experimental.pallas import tpu_sc as plsc

D_MODEL = 64
NUM_CORES = 2
NUM_SUBCORES = 16
NUM_WORKERS = NUM_CORES * NUM_SUBCORES
CHUNK = 512


def _emb_body(idx_hbm, table_hbm, out_hbm, idx_v, rows_v, sem, *, b_per_w):
    c = lax.axis_index("c")
    s = lax.axis_index("s")
    wid = s * NUM_CORES + c
    base = wid * b_per_w
    n_chunks = b_per_w // CHUNK

    def chunk_body(i, carry):
        off = base + i * CHUNK
        pltpu.sync_copy(idx_hbm.at[pl.ds(off, CHUNK)], idx_v)
        pltpu.async_copy(table_hbm.at[idx_v], rows_v, sem).wait()
        pltpu.sync_copy(rows_v, out_hbm.at[pl.ds(off, CHUNK)])
        return carry

    lax.fori_loop(0, n_chunks, chunk_body, 0)


def kernel(input, weight):
    b, t = input.shape
    n = b * t
    assert n % (NUM_WORKERS * CHUNK) == 0
    b_per_w = n // NUM_WORKERS
    idx = input.reshape(n).astype(jnp.int32)

    body = functools.partial(_emb_body, b_per_w=b_per_w)
    mesh = plsc.VectorSubcoreMesh(core_axis_name="c", subcore_axis_name="s")
    out = pl.kernel(
        body,
        out_type=jax.ShapeDtypeStruct((n, D_MODEL), jnp.float32),
        mesh=mesh,
        compiler_params=pltpu.CompilerParams(use_tc_tiling_on_sc=False),
        scratch_types=[
            pltpu.VMEM((CHUNK,), jnp.int32),
            pltpu.VMEM((CHUNK, D_MODEL), jnp.float32),
            pltpu.SemaphoreType.DMA,
        ],
    )(idx, weight)
    return out.reshape(b, t, D_MODEL)


# speedup vs baseline: 1.0475x; 1.0475x over previous
import functools

import jax
import jax.numpy as jnp
from jax import lax
from jax.experimental import pallas as pl
from jax.experimental.pallas import tpu as pltpu
from jax.experimental.pallas import tpu_sc as plsc

D_MODEL = 64
NUM_CORES = 2
NUM_SUBCORES = 16
NUM_WORKERS = NUM_CORES * NUM_SUBCORES
CHUNK = 512
NBUF = 2


def _emb_body(idx_hbm, table_hbm, out_hbm, idx_v, rows, gsems, ssems, *,
              b_per_w):
    c = lax.axis_index("c")
    s = lax.axis_index("s")
    wid = s * NUM_CORES + c
    base = wid * b_per_w
    n_chunks = b_per_w // CHUNK

    pltpu.sync_copy(idx_hbm.at[pl.ds(base, b_per_w)], idx_v)

    def start_gather(g, b):
        pltpu.async_copy(
            table_hbm.at[idx_v.at[pl.ds(g * CHUNK, CHUNK)]], rows[b], gsems[b])

    def start_store(g, b):
        pltpu.async_copy(rows[b], out_hbm.at[pl.ds(base + g * CHUNK, CHUNK)],
                         ssems[b])

    def wait_gather(b):
        pltpu.make_async_copy(table_hbm.at[pl.ds(0, CHUNK)], rows[b],
                              gsems[b]).wait()

    def wait_store(b):
        pltpu.make_async_copy(rows[b], out_hbm.at[pl.ds(0, CHUNK)],
                              ssems[b]).wait()

    for b in range(NBUF):
        start_gather(b, b)

    @pl.loop(0, n_chunks - NBUF, step=NBUF)
    def _(k):
        for b in range(NBUF):
            g = k + b
            wait_gather(b)
            start_store(g, b)
            wait_store(b)
            start_gather(g + NBUF, b)

    for b in range(NBUF):
        wait_gather(b)
        start_store(n_chunks - NBUF + b, b)
    for b in range(NBUF):
        wait_store(b)


def kernel(input, weight):
    b, t = input.shape
    n = b * t
    assert n % (NUM_WORKERS * CHUNK) == 0
    b_per_w = n // NUM_WORKERS
    idx = input.reshape(n).astype(jnp.int32)

    body = functools.partial(_emb_body, b_per_w=b_per_w)
    mesh = plsc.VectorSubcoreMesh(core_axis_name="c", subcore_axis_name="s")
    out = pl.kernel(
        body,
        out_type=jax.ShapeDtypeStruct((n, D_MODEL), jnp.float32),
        mesh=mesh,
        compiler_params=pltpu.CompilerParams(use_tc_tiling_on_sc=False),
        scratch_types=[
            pltpu.VMEM((b_per_w,), jnp.int32),
            [pltpu.VMEM((CHUNK, D_MODEL), jnp.float32) for _ in range(NBUF)],
            [pltpu.SemaphoreType.DMA for _ in range(NBUF)],
            [pltpu.SemaphoreType.DMA for _ in range(NBUF)],
        ],
    )(idx, weight)
    return out.reshape(b, t, D_MODEL)


# speedup vs baseline: 1.0484x; 1.0009x over previous
import functools

import jax
import jax.numpy as jnp
from jax import lax
from jax.experimental import pallas as pl
from jax.experimental.pallas import tpu as pltpu
from jax.experimental.pallas import tpu_sc as plsc

D_MODEL = 64
NUM_CORES = 2
NUM_SUBCORES = 16
NUM_WORKERS = NUM_CORES * NUM_SUBCORES
CHUNK = 512
NBUF = 2


def _emb_body(idx_hbm, table_hbm, out_hbm, idx_v, rows, gsems, ssems, *,
              b_per_w):
    c = lax.axis_index("c")
    s = lax.axis_index("s")
    wid = s * NUM_CORES + c
    base = wid * b_per_w
    n_chunks = b_per_w // CHUNK

    pltpu.sync_copy(idx_hbm.at[pl.ds(base, b_per_w)], idx_v)

    def start_gather(g, b):
        pltpu.async_copy(
            table_hbm.at[idx_v.at[pl.ds(g * CHUNK, CHUNK)]], rows[b], gsems[b])

    def start_store(g, b):
        pltpu.async_copy(rows[b], out_hbm.at[pl.ds(base + g * CHUNK, CHUNK)],
                         ssems[b])

    def wait_gather(b):
        pltpu.make_async_copy(table_hbm.at[pl.ds(0, CHUNK)], rows[b],
                              gsems[b]).wait()

    def wait_store(b):
        pltpu.make_async_copy(rows[b], out_hbm.at[pl.ds(0, CHUNK)],
                              ssems[b]).wait()

    for b in range(NBUF):
        start_gather(b, b)

    @pl.loop(0, n_chunks - NBUF, step=NBUF)
    def _(k):
        for b in range(NBUF):
            g = k + b
            wait_gather(b)
            start_store(g, b)
            wait_store(b)
            start_gather(g + NBUF, b)

    for b in range(NBUF):
        wait_gather(b)
        start_store(n_chunks - NBUF + b, b)
    for b in range(NBUF):
        wait_store(b)


def kernel(input, weight):
    b, t = input.shape
    n = b * t
    assert n % (NUM_WORKERS * CHUNK) == 0
    b_per_w = n // NUM_WORKERS
    idx = input.reshape(n).astype(jnp.int32)

    body = functools.partial(_emb_body, b_per_w=b_per_w)
    mesh = plsc.VectorSubcoreMesh(core_axis_name="c", subcore_axis_name="s")
    out = pl.kernel(
        body,
        out_type=jax.ShapeDtypeStruct((n, D_MODEL), jnp.float32),
        mesh=mesh,
        compiler_params=pltpu.CompilerParams(
            use_tc_tiling_on_sc=False,
            skip_device_barrier=True,
            disable_bounds_checks=True,
            disable_semaphore_checks=True,
        ),
        scratch_types=[
            pltpu.VMEM((b_per_w,), jnp.int32),
            [pltpu.VMEM((CHUNK, D_MODEL), jnp.float32) for _ in range(NBUF)],
            [pltpu.SemaphoreType.DMA for _ in range(NBUF)],
            [pltpu.SemaphoreType.DMA for _ in range(NBUF)],
        ],
    )(idx, weight)
    return out.reshape(b, t, D_MODEL)


# speedup vs baseline: 1.3947x; 1.3303x over previous
import functools

import jax
import jax.numpy as jnp
from jax import lax
from jax.experimental import pallas as pl
from jax.experimental.pallas import tpu as pltpu
from jax.experimental.pallas import tpu_sc as plsc

D_MODEL = 64
D_PAD = 128
NUM_CORES = 2
NUM_SUBCORES = 16
NUM_WORKERS = NUM_CORES * NUM_SUBCORES
CHUNK = 256
NBUF = 2


def _emb_body(idx_hbm, table_hbm, out_hbm, idx_v, stg, gsems, ssems, *,
              b_per_w):
    c = lax.axis_index("c")
    s = lax.axis_index("s")
    wid = s * NUM_CORES + c
    base = wid * b_per_w
    n_chunks = b_per_w // CHUNK

    pltpu.sync_copy(idx_hbm.at[pl.ds(base, b_per_w)], idx_v)

    def start_gather(g, b):
        pltpu.async_copy(
            table_hbm.at[idx_v.at[pl.ds(g * CHUNK, CHUNK)]], stg[b], gsems[b])

    def start_store(g, b):
        pltpu.async_copy(
            stg[b],
            out_hbm.at[pl.ds(base + g * CHUNK, CHUNK), pl.ds(0, D_MODEL)],
            ssems[b])

    def wait_gather(b):
        pltpu.make_async_copy(table_hbm.at[pl.ds(0, CHUNK)], stg[b],
                              gsems[b]).wait()

    def wait_store(b):
        pltpu.make_async_copy(stg[b],
                              out_hbm.at[pl.ds(0, CHUNK), pl.ds(0, D_MODEL)],
                              ssems[b]).wait()

    for b in range(NBUF):
        start_gather(b, b)

    @pl.loop(0, n_chunks - NBUF, step=NBUF)
    def _(k):
        for b in range(NBUF):
            g = k + b
            wait_gather(b)
            start_store(g, b)
            wait_store(b)
            start_gather(g + NBUF, b)

    for b in range(NBUF):
        wait_gather(b)
        start_store(n_chunks - NBUF + b, b)
    for b in range(NBUF):
        wait_store(b)


def kernel(input, weight):
    b, t = input.shape
    n = b * t
    assert n % (NUM_WORKERS * CHUNK) == 0
    b_per_w = n // NUM_WORKERS
    idx = input.reshape(n).astype(jnp.int32)

    body = functools.partial(_emb_body, b_per_w=b_per_w)
    mesh = plsc.VectorSubcoreMesh(core_axis_name="c", subcore_axis_name="s")
    out = pl.kernel(
        body,
        out_type=jax.ShapeDtypeStruct((n, D_PAD), jnp.float32),
        mesh=mesh,
        compiler_params=pltpu.CompilerParams(
            use_tc_tiling_on_sc=False,
            skip_device_barrier=True,
            disable_bounds_checks=True,
            disable_semaphore_checks=True,
        ),
        scratch_types=[
            pltpu.VMEM((b_per_w,), jnp.int32),
            [pltpu.VMEM((CHUNK, D_MODEL), jnp.float32) for _ in range(NBUF)],
            [pltpu.SemaphoreType.DMA for _ in range(NBUF)],
            [pltpu.SemaphoreType.DMA for _ in range(NBUF)],
        ],
    )(idx, weight)
    return out[:, :D_MODEL].reshape(b, t, D_MODEL)
